# baseline (speedup 1.0000x reference)
import functools

import numpy as np

import jax
import jax.numpy as jnp
from jax import lax
from jax.experimental import pallas as pl
from jax.experimental.pallas import tpu as pltpu
from jax.experimental.pallas import tpu_sc as plsc

E = 128
LSEQ = 200
B = 1024
K = 4
KP = K + 1
N = B * LSEQ
NW = 32
TPW = N // NW
T = 64
NCHUNK = TPW // T
NITER = NCHUNK // 4

_PERM = (np.arange(4)[:, None] * 32
         + np.stack([np.arange(16), np.arange(16) + 16], 1).reshape(32)[None, :]
         ).reshape(128)


def _pack_table(tab):
    t = tab[:, _PERM].astype(jnp.bfloat16)
    return jax.lax.bitcast_convert_type(t.reshape(-1, 64, 2), jnp.int32)


def _body(seq_hbm, cseq_hbm, tok_tab, cont_tab, pe_hbm, out_hbm,
          pe_v, t0, t1, t2, t3, c0, c1, c2, c3, a0, a1, a2, a3,
          m0, m1, m2, m3,
          gs0, gs1, gs2, gs3, os0, os1, os2, os3, is0, is1, is2, is3):
    c = lax.axis_index("c")
    s = lax.axis_index("s")
    wid = s * 2 + c

    tis = (t0, t1, t2, t3)
    cis = (c0, c1, c2, c3)
    accs = (a0, a1, a2, a3)
    tmps = (m0, m1, m2, m3)
    semG = (gs0, gs1, gs2, gs3)
    semO = (os0, os1, os2, os3)
    semI = (is0, is1, is2, is3)

    pltpu.sync_copy(pe_hbm, pe_v)

    def idx_copies(slot, n):
        return [pltpu.make_async_copy(seq_hbm.at[wid, n], tis[slot],
                                      semI[slot]),
                pltpu.make_async_copy(cseq_hbm.at[wid, n], cis[slot],
                                      semI[slot])]

    def gathers(slot):
        cps = [pltpu.make_async_copy(tok_tab.at[tis[slot]], accs[slot],
                                     semG[slot])]
        for h in range(2):
            cps.append(pltpu.make_async_copy(
                cont_tab.at[cis[slot].at[h]],
                tmps[slot].at[pl.ds(h * 128, 128)], semG[slot]))
        return cps

    def out_copy(slot, base):
        return pltpu.make_async_copy(accs[slot], out_hbm.at[pl.ds(base, T)],
                                     semO[slot])

    for j in range(4):
        for cp in idx_copies(j, j):
            cp.start()
    for j in range(2):
        for cp in idx_copies(j, j):
            cp.wait()
        for cp in gathers(j):
            cp.start()

    def compute(slot, n):
        acc = accs[slot]
        tm = tmps[slot]
        l0 = lax.rem(n * T, LSEQ)

        def unpk(w):
            a = lax.bitcast_convert_type(w << 16, jnp.float32)
            b = lax.bitcast_convert_type(w & jnp.int32(-65536), jnp.float32)
            return a, b

        def row(i):
            li = l0 + i
            li = jnp.where(li >= LSEQ, li - LSEQ, li)
            for dg in range(4):
                wsl = pl.ds(16 * dg, 16)
                va = None
                vb = None
                for k in range(K):
                    a, b = unpk(tm[4 * i + k, wsl])
                    va = a if va is None else va + a
                    vb = b if vb is None else vb + b
                ap, bp = unpk(pe_v[li, wsl])
                va = va + ap
                vb = vb + bp
                plsc.addupdate(acc.at[i, pl.ds(32 * dg, 16)], va)
                plsc.addupdate(acc.at[i, pl.ds(32 * dg + 16, 16)], vb)

        def rowpair(t, _):
            row(2 * t)
            row(2 * t + 1)
            return 0

        lax.fori_loop(0, T // 2, rowpair, 0)

    def chunk(u, m):
        n = 4 * m + u
        base = wid * TPW + n * T
        for cp in gathers(u):
            cp.wait()
        compute(u, n)
        out_copy(u, base).start()

        r = (u + 2) % 4

        def refill():
            for cp in idx_copies(r, n + 2):
                cp.wait()
            for cp in gathers(r):
                cp.start()

        def issue_idx():
            for cp in idx_copies(u, n + 4):
                cp.start()

        if u < 2:
            @pl.when(m > 0)
            def _():
                out_copy(r, base).wait()
            refill()

            @pl.when(m < NITER - 1)
            def _():
                issue_idx()
        else:
            @pl.when(m < NITER - 1)
            def _():
                out_copy(r, base).wait()
                refill()
                issue_idx()

    def body(m, _):
        for u in range(4):
            chunk(u, m)
        return 0

    lax.fori_loop(0, NITER, body, 0)

    for u in range(4):
        out_copy(u, wid * TPW).wait()


@jax.jit
def _run(seq_r, cseq_r, token_table, cont_packed, pe_packed):
    kern = pl.kernel(
        _body,
        out_type=jax.ShapeDtypeStruct((N, E), jnp.float32),
        mesh=plsc.VectorSubcoreMesh(core_axis_name="c", subcore_axis_name="s"),
        compiler_params=pltpu.CompilerParams(use_tc_tiling_on_sc=False),
        scratch_types=(
            [pltpu.VMEM((LSEQ, E // 2), jnp.int32)]
            + [pltpu.VMEM((T,), jnp.int32)] * 4
            + [pltpu.VMEM((2, 2 * T), jnp.int32)] * 4
            + [pltpu.VMEM((T, E), jnp.float32)] * 4
            + [pltpu.VMEM((K * T, E // 2), jnp.int32)] * 4
            + [pltpu.SemaphoreType.DMA] * 12
        ),
    )
    return kern(seq_r, cseq_r, token_table, cont_packed, pe_packed)


def kernel(sequence, c_sequence, token_table, content_table, pe):
    seq_r = sequence.astype(jnp.int32).reshape(NW, NCHUNK, T)
    cseq_r = c_sequence.astype(jnp.int32).reshape(NW, NCHUNK, 2, 2 * T)
    cont_packed = _pack_table(content_table)
    pe_packed = _pack_table(pe[0, :LSEQ])
    out = _run(seq_r, cseq_r, token_table, cont_packed, pe_packed)
    return out.reshape(B, LSEQ, E)


# speedup vs baseline: 1.0013x; 1.0013x over previous
import functools

import numpy as np

import jax
import jax.numpy as jnp
from jax import lax
from jax.experimental import pallas as pl
from jax.experimental.pallas import tpu as pltpu
from jax.experimental.pallas import tpu_sc as plsc

E = 128
LSEQ = 200
B = 1024
K = 4
KP = K + 1
N = B * LSEQ
NW = 32
TPW = N // NW
T = 64
NCHUNK = TPW // T
NITER = NCHUNK // 4

_PERM = (np.arange(4)[:, None] * 32
         + np.stack([np.arange(16), np.arange(16) + 16], 1).reshape(32)[None, :]
         ).reshape(128)


def _pack_table(tab):
    t = tab[:, _PERM].astype(jnp.bfloat16)
    return jax.lax.bitcast_convert_type(t.reshape(-1, 64, 2), jnp.int32)


def _body(seq_hbm, cseq_hbm, tok_tab, cont_tab, pe_hbm, out_hbm,
          pe_v, t0, t1, t2, t3, c0, c1, c2, c3, a0, a1, a2, a3,
          m0, m1, m2, m3,
          gs0, gs1, gs2, gs3, os0, os1, os2, os3, is0, is1, is2, is3):
    c = lax.axis_index("c")
    s = lax.axis_index("s")
    wid = s * 2 + c

    tis = (t0, t1, t2, t3)
    cis = (c0, c1, c2, c3)
    accs = (a0, a1, a2, a3)
    tmps = (m0, m1, m2, m3)
    semG = (gs0, gs1, gs2, gs3)
    semO = (os0, os1, os2, os3)
    semI = (is0, is1, is2, is3)

    pltpu.sync_copy(pe_hbm, pe_v)

    def idx_copies(slot, n):
        return [pltpu.make_async_copy(seq_hbm.at[wid, n], tis[slot],
                                      semI[slot]),
                pltpu.make_async_copy(cseq_hbm.at[wid, n], cis[slot],
                                      semI[slot])]

    def gathers(slot):
        cps = [pltpu.make_async_copy(tok_tab.at[tis[slot]], accs[slot],
                                     semG[slot])]
        for h in range(4):
            cps.append(pltpu.make_async_copy(
                cont_tab.at[cis[slot].at[h]],
                tmps[slot].at[pl.ds(h * 64, 64)], semG[slot]))
        return cps

    def out_copy(slot, base):
        return pltpu.make_async_copy(accs[slot], out_hbm.at[pl.ds(base, T)],
                                     semO[slot])

    for j in range(4):
        for cp in idx_copies(j, j):
            cp.start()
    for j in range(2):
        for cp in idx_copies(j, j):
            cp.wait()
        for cp in gathers(j):
            cp.start()

    def compute(slot, n):
        acc = accs[slot]
        tm = tmps[slot]
        l0 = lax.rem(n * T, LSEQ)

        def unpk(w):
            a = lax.bitcast_convert_type(w << 16, jnp.float32)
            b = lax.bitcast_convert_type(w & jnp.int32(-65536), jnp.float32)
            return a, b

        def row(i):
            li = l0 + i
            li = jnp.where(li >= LSEQ, li - LSEQ, li)
            for dg in range(4):
                wsl = pl.ds(16 * dg, 16)
                va = None
                vb = None
                for k in range(K):
                    a, b = unpk(tm[4 * i + k, wsl])
                    va = a if va is None else va + a
                    vb = b if vb is None else vb + b
                ap, bp = unpk(pe_v[li, wsl])
                va = va + ap
                vb = vb + bp
                plsc.addupdate(acc.at[i, pl.ds(32 * dg, 16)], va)
                plsc.addupdate(acc.at[i, pl.ds(32 * dg + 16, 16)], vb)

        def rowpair(t, _):
            row(2 * t)
            row(2 * t + 1)
            return 0

        lax.fori_loop(0, T // 2, rowpair, 0)

    def chunk(u, m):
        n = 4 * m + u
        base = wid * TPW + n * T
        for cp in gathers(u):
            cp.wait()
        compute(u, n)
        out_copy(u, base).start()

        r = (u + 2) % 4

        def refill():
            for cp in idx_copies(r, n + 2):
                cp.wait()
            for cp in gathers(r):
                cp.start()

        def issue_idx():
            for cp in idx_copies(u, n + 4):
                cp.start()

        if u < 2:
            @pl.when(m > 0)
            def _():
                out_copy(r, base).wait()
            refill()

            @pl.when(m < NITER - 1)
            def _():
                issue_idx()
        else:
            @pl.when(m < NITER - 1)
            def _():
                out_copy(r, base).wait()
                refill()
                issue_idx()

    def body(m, _):
        for u in range(4):
            chunk(u, m)
        return 0

    lax.fori_loop(0, NITER, body, 0)

    for u in range(4):
        out_copy(u, wid * TPW).wait()


@jax.jit
def _run(seq_r, cseq_r, token_table, cont_packed, pe_packed):
    kern = pl.kernel(
        _body,
        out_type=jax.ShapeDtypeStruct((N, E), jnp.float32),
        mesh=plsc.VectorSubcoreMesh(core_axis_name="c", subcore_axis_name="s"),
        compiler_params=pltpu.CompilerParams(use_tc_tiling_on_sc=False),
        scratch_types=(
            [pltpu.VMEM((LSEQ, E // 2), jnp.int32)]
            + [pltpu.VMEM((T,), jnp.int32)] * 4
            + [pltpu.VMEM((4, T), jnp.int32)] * 4
            + [pltpu.VMEM((T, E), jnp.float32)] * 4
            + [pltpu.VMEM((K * T, E // 2), jnp.int32)] * 4
            + [pltpu.SemaphoreType.DMA] * 12
        ),
    )
    return kern(seq_r, cseq_r, token_table, cont_packed, pe_packed)


def kernel(sequence, c_sequence, token_table, content_table, pe):
    seq_r = sequence.astype(jnp.int32).reshape(NW, NCHUNK, T)
    cseq_r = c_sequence.astype(jnp.int32).reshape(NW, NCHUNK, 4, T)
    cont_packed = _pack_table(content_table)
    pe_packed = _pack_table(pe[0, :LSEQ])
    out = _run(seq_r, cseq_r, token_table, cont_packed, pe_packed)
    return out.reshape(B, LSEQ, E)


# speedup vs baseline: 1.0265x; 1.0252x over previous
import functools

import numpy as np

import jax
import jax.numpy as jnp
from jax import lax
from jax.experimental import pallas as pl
from jax.experimental.pallas import tpu as pltpu
from jax.experimental.pallas import tpu_sc as plsc

E = 128
LSEQ = 200
B = 1024
K = 4
KP = K + 1
N = B * LSEQ
NW = 32
TPW = N // NW
T = 64
NCHUNK = TPW // T
NITER = NCHUNK // 4

_PERM = (np.arange(4)[:, None] * 32
         + np.stack([np.arange(16), np.arange(16) + 16], 1).reshape(32)[None, :]
         ).reshape(128)


def _pack_table(tab):
    t = tab[:, _PERM].astype(jnp.bfloat16)
    return jax.lax.bitcast_convert_type(t.reshape(-1, 64, 2), jnp.int32)


def _body(idxc_hbm, tok_tab, cont_tab, pe_hbm, out_hbm,
          pe_v, i0, i1, i2, i3, a0, a1, a2, a3,
          m0, m1, m2, m3,
          gs0, gs1, gs2, gs3, os0, os1, os2, os3, is0, is1, is2, is3):
    c = lax.axis_index("c")
    s = lax.axis_index("s")
    wid = s * 2 + c

    idxs = (i0, i1, i2, i3)
    accs = (a0, a1, a2, a3)
    tmps = (m0, m1, m2, m3)
    semG = (gs0, gs1, gs2, gs3)
    semO = (os0, os1, os2, os3)
    semI = (is0, is1, is2, is3)

    pltpu.sync_copy(pe_hbm, pe_v)

    def idx_copies(slot, n):
        return [pltpu.make_async_copy(idxc_hbm.at[wid, n], idxs[slot],
                                      semI[slot])]

    def gathers(slot):
        cps = [pltpu.make_async_copy(tok_tab.at[idxs[slot].at[0]], accs[slot],
                                     semG[slot])]
        for h in range(4):
            cps.append(pltpu.make_async_copy(
                cont_tab.at[idxs[slot].at[h + 1]],
                tmps[slot].at[pl.ds(h * 64, 64)], semG[slot]))
        return cps

    def out_copy(slot, base):
        return pltpu.make_async_copy(accs[slot], out_hbm.at[pl.ds(base, T)],
                                     semO[slot])

    for j in range(4):
        for cp in idx_copies(j, j):
            cp.start()
    for j in range(2):
        for cp in idx_copies(j, j):
            cp.wait()
        for cp in gathers(j):
            cp.start()

    def compute(slot, n):
        acc = accs[slot]
        tm = tmps[slot]
        l0 = lax.rem(n * T, LSEQ)

        def unpk(w):
            a = lax.bitcast_convert_type(w << 16, jnp.float32)
            b = lax.bitcast_convert_type(w & jnp.int32(-65536), jnp.float32)
            return a, b

        def row(i):
            li = l0 + i
            li = jnp.where(li >= LSEQ, li - LSEQ, li)
            for dg in range(4):
                wsl = pl.ds(16 * dg, 16)
                va = None
                vb = None
                for k in range(K):
                    a, b = unpk(tm[4 * i + k, wsl])
                    va = a if va is None else va + a
                    vb = b if vb is None else vb + b
                ap, bp = unpk(pe_v[li, wsl])
                va = va + ap
                vb = vb + bp
                plsc.addupdate(acc.at[i, pl.ds(32 * dg, 16)], va)
                plsc.addupdate(acc.at[i, pl.ds(32 * dg + 16, 16)], vb)

        def rowpair(t, _):
            row(2 * t)
            row(2 * t + 1)
            return 0

        lax.fori_loop(0, T // 2, rowpair, 0)

    def chunk(u, m):
        n = 4 * m + u
        base = wid * TPW + n * T
        for cp in gathers(u):
            cp.wait()
        compute(u, n)
        out_copy(u, base).start()

        r = (u + 2) % 4

        def refill():
            for cp in idx_copies(r, n + 2):
                cp.wait()
            for cp in gathers(r):
                cp.start()

        def issue_idx():
            for cp in idx_copies(u, n + 4):
                cp.start()

        if u < 2:
            @pl.when(m > 0)
            def _():
                out_copy(r, base).wait()
            refill()

            @pl.when(m < NITER - 1)
            def _():
                issue_idx()
        else:
            @pl.when(m < NITER - 1)
            def _():
                out_copy(r, base).wait()
                refill()
                issue_idx()

    def body(m, _):
        for u in range(4):
            chunk(u, m)
        return 0

    lax.fori_loop(0, NITER, body, 0)

    for u in range(4):
        out_copy(u, wid * TPW).wait()


@jax.jit
def _run(idxc, token_table, cont_packed, pe_packed):
    kern = pl.kernel(
        _body,
        out_type=jax.ShapeDtypeStruct((N, E), jnp.float32),
        mesh=plsc.VectorSubcoreMesh(core_axis_name="c", subcore_axis_name="s"),
        compiler_params=pltpu.CompilerParams(use_tc_tiling_on_sc=False),
        scratch_types=(
            [pltpu.VMEM((LSEQ, E // 2), jnp.int32)]
            + [pltpu.VMEM((KP, T), jnp.int32)] * 4
            + [pltpu.VMEM((T, E), jnp.float32)] * 4
            + [pltpu.VMEM((K * T, E // 2), jnp.int32)] * 4
            + [pltpu.SemaphoreType.DMA] * 12
        ),
    )
    return kern(idxc, token_table, cont_packed, pe_packed)


def kernel(sequence, c_sequence, token_table, content_table, pe):
    tok = sequence.astype(jnp.int32).reshape(NW, NCHUNK, 1, T)
    con = c_sequence.astype(jnp.int32).reshape(NW, NCHUNK, K, T)
    idxc = jnp.concatenate([tok, con], axis=2)
    cont_packed = _pack_table(content_table)
    pe_packed = _pack_table(pe[0, :LSEQ])
    out = _run(idxc, token_table, cont_packed, pe_packed)
    return out.reshape(B, LSEQ, E)


# speedup vs baseline: 1.2355x; 1.2035x over previous
import functools

import numpy as np

import jax
import jax.numpy as jnp
from jax import lax
from jax.experimental import pallas as pl
from jax.experimental.pallas import tpu as pltpu
from jax.experimental.pallas import tpu_sc as plsc

E = 128
LSEQ = 200
B = 1024
K = 4
KP = K + 1
N = B * LSEQ
NW = 32
TPW = N // NW
T = 64
NCHUNK = TPW // T
NITER = NCHUNK // 4

_PERM = (np.arange(4)[:, None] * 32
         + np.stack([np.arange(16), np.arange(16) + 16], 1).reshape(32)[None, :]
         ).reshape(128)


def _pack_table(tab):
    t = tab[:, _PERM].astype(jnp.bfloat16)
    return jax.lax.bitcast_convert_type(t.reshape(-1, 64, 2), jnp.int32)


def _body(idxc_hbm, tok_tab, cont_hbm, pe_hbm, out_hbm,
          pe_v, cont_v, i0, i1, i2, i3, a0, a1, a2, a3,
          gs0, gs1, gs2, gs3, os0, os1, os2, os3, is0, is1, is2, is3):
    c = lax.axis_index("c")
    s = lax.axis_index("s")
    wid = s * 2 + c

    idxs = (i0, i1, i2, i3)
    accs = (a0, a1, a2, a3)
    semG = (gs0, gs1, gs2, gs3)
    semO = (os0, os1, os2, os3)
    semI = (is0, is1, is2, is3)

    pltpu.sync_copy(pe_hbm, pe_v)
    pltpu.sync_copy(cont_hbm, cont_v)

    def idx_copies(slot, n):
        return [pltpu.make_async_copy(idxc_hbm.at[wid, n], idxs[slot],
                                      semI[slot])]

    def gathers(slot):
        return [pltpu.make_async_copy(tok_tab.at[idxs[slot].at[0]], accs[slot],
                                      semG[slot])]

    def out_copy(slot, base):
        return pltpu.make_async_copy(accs[slot], out_hbm.at[pl.ds(base, T)],
                                     semO[slot])

    for j in range(4):
        for cp in idx_copies(j, j):
            cp.start()
    for j in range(2):
        for cp in idx_copies(j, j):
            cp.wait()
        for cp in gathers(j):
            cp.start()

    _IOTA = lax.iota(jnp.int32, 16)

    def compute(slot, n):
        acc = accs[slot]
        idx = idxs[slot]
        l0 = lax.rem(n * T, LSEQ)

        def unpk(w):
            return plsc.unpack(plsc.bitcast(w, jnp.bfloat16),
                               format=plsc.PackFormat.INTERLEAVED)

        def row(i):
            li = l0 + i
            li = jnp.where(li >= LSEQ, li - LSEQ, li)
            ivec = jnp.full((16,), i, jnp.int32)
            base = [plsc.load_gather(idx, [jnp.full((16,), k + 1, jnp.int32),
                                           ivec]) << 6
                    for k in range(K)]
            for dg in range(4):
                col = _IOTA + 16 * dg
                va = None
                vb = None
                for k in range(K):
                    a, b = unpk(plsc.load_gather(cont_v, [base[k] + col]))
                    va = a if va is None else va + a
                    vb = b if vb is None else vb + b
                ap, bp = unpk(pe_v[li, pl.ds(16 * dg, 16)])
                va = va + ap
                vb = vb + bp
                plsc.addupdate(acc.at[i, pl.ds(32 * dg, 16)], va)
                plsc.addupdate(acc.at[i, pl.ds(32 * dg + 16, 16)], vb)

        def rowquad(t, _):
            for j in range(4):
                row(4 * t + j)
            return 0

        lax.fori_loop(0, T // 4, rowquad, 0)

    def chunk(u, m):
        n = 4 * m + u
        base = wid * TPW + n * T
        for cp in gathers(u):
            cp.wait()
        compute(u, n)
        out_copy(u, base).start()

        r = (u + 2) % 4

        def refill():
            for cp in idx_copies(r, n + 2):
                cp.wait()
            for cp in gathers(r):
                cp.start()

        def issue_idx():
            for cp in idx_copies(u, n + 4):
                cp.start()

        if u < 2:
            @pl.when(m > 0)
            def _():
                out_copy(r, base).wait()
            refill()

            @pl.when(m < NITER - 1)
            def _():
                issue_idx()
        else:
            @pl.when(m < NITER - 1)
            def _():
                out_copy(r, base).wait()
                refill()
                issue_idx()

    def body(m, _):
        for u in range(4):
            chunk(u, m)
        return 0

    lax.fori_loop(0, NITER, body, 0)

    for u in range(4):
        out_copy(u, wid * TPW).wait()


@jax.jit
def _run(idxc, token_table, cont_packed, pe_packed):
    kern = pl.kernel(
        _body,
        out_type=jax.ShapeDtypeStruct((N, E), jnp.float32),
        mesh=plsc.VectorSubcoreMesh(core_axis_name="c", subcore_axis_name="s"),
        compiler_params=pltpu.CompilerParams(use_tc_tiling_on_sc=False,
                                             needs_layout_passes=False),
        scratch_types=(
            [pltpu.VMEM((LSEQ, E // 2), jnp.int32)]
            + [pltpu.VMEM((1000 * E // 2,), jnp.int32)]
            + [pltpu.VMEM((KP, T), jnp.int32)] * 4
            + [pltpu.VMEM((T, E), jnp.float32)] * 4
            + [pltpu.SemaphoreType.DMA] * 12
        ),
    )
    return kern(idxc, token_table, cont_packed, pe_packed)


def kernel(sequence, c_sequence, token_table, content_table, pe):
    tok = sequence.astype(jnp.int32).reshape(NW, NCHUNK, 1, T)
    con = (c_sequence.astype(jnp.int32)
           .reshape(NW, NCHUNK, T, K).transpose(0, 1, 3, 2))
    idxc = jnp.concatenate([tok, con], axis=2)
    cont_packed = _pack_table(content_table).reshape(-1)
    pe_packed = _pack_table(pe[0, :LSEQ])
    out = _run(idxc, token_table, cont_packed, pe_packed)
    return out.reshape(B, LSEQ, E)


# speedup vs baseline: 1.2830x; 1.0384x over previous
import functools

import numpy as np

import jax
import jax.numpy as jnp
from jax import lax
from jax.experimental import pallas as pl
from jax.experimental.pallas import tpu as pltpu
from jax.experimental.pallas import tpu_sc as plsc

E = 128
LSEQ = 200
B = 1024
K = 4
KP = K + 1
N = B * LSEQ
NW = 32
TPW = N // NW
T = 64
NCHUNK = TPW // T
NITER = NCHUNK // 4

_PERM = (np.arange(4)[:, None] * 32
         + np.stack([np.arange(16), np.arange(16) + 16], 1).reshape(32)[None, :]
         ).reshape(128)


def _pack_table(tab):
    t = tab[:, _PERM].astype(jnp.bfloat16)
    return jax.lax.bitcast_convert_type(t.reshape(-1, 64, 2), jnp.int32)


def _body(idxc_hbm, tok_tab, cont_tab, pe_hbm, out_hbm,
          pe_v, i0, i1, i2, i3, a0, a1, a2, a3, m0, m1, m2, m3,
          gs0, gs1, gs2, gs3, os0, os1, os2, os3, is0, is1, is2, is3):
    c = lax.axis_index("c")
    s = lax.axis_index("s")
    wid = s * 2 + c

    idxs = (i0, i1, i2, i3)
    accs = (a0, a1, a2, a3)
    tmps = (m0, m1, m2, m3)
    semG = (gs0, gs1, gs2, gs3)
    semO = (os0, os1, os2, os3)
    semI = (is0, is1, is2, is3)

    pltpu.sync_copy(pe_hbm, pe_v)

    def idx_copy(slot, n):
        return pltpu.make_async_copy(idxc_hbm.at[wid, n], idxs[slot], semI[slot])

    def gathers(slot):
        cps = [pltpu.make_async_copy(tok_tab.at[idxs[slot].at[0]], accs[slot],
                                     semG[slot])]
        for k in range(K):
            cps.append(pltpu.make_async_copy(
                cont_tab.at[idxs[slot].at[k + 1]],
                tmps[slot].at[k], semG[slot]))
        return cps

    def out_copy(slot, base):
        return pltpu.make_async_copy(accs[slot], out_hbm.at[pl.ds(base, T)],
                                     semO[slot])

    for j in range(4):
        idx_copy(j, j).start()
    for j in range(2):
        idx_copy(j, j).wait()
        for cp in gathers(j):
            cp.start()

    def compute(slot, n):
        acc = accs[slot]
        tm = tmps[slot]
        l0 = lax.rem(n * T, LSEQ)

        def unpk(w):
            a = lax.bitcast_convert_type(w << 16, jnp.float32)
            b = lax.bitcast_convert_type(w & jnp.int32(-65536), jnp.float32)
            return a, b

        def row(i):
            li = l0 + i
            li = jnp.where(li >= LSEQ, li - LSEQ, li)
            for dg in range(4):
                wsl = pl.ds(16 * dg, 16)
                va = None
                vb = None
                for k in range(K):
                    a, b = unpk(tm[k, i, wsl])
                    va = a if va is None else va + a
                    vb = b if vb is None else vb + b
                ap, bp = unpk(pe_v[li, wsl])
                va = va + ap
                vb = vb + bp
                plsc.addupdate(acc.at[i, pl.ds(32 * dg, 16)], va)
                plsc.addupdate(acc.at[i, pl.ds(32 * dg + 16, 16)], vb)

        def rowpair(t, _):
            row(2 * t)
            row(2 * t + 1)
            return 0

        lax.fori_loop(0, T // 2, rowpair, 0)

    def chunk(u, m):
        n = 4 * m + u
        base = wid * TPW + n * T
        for cp in gathers(u):
            cp.wait()
        compute(u, n)
        out_copy(u, base).start()

        r = (u + 2) % 4

        def refill():
            idx_copy(r, n + 2).wait()
            for cp in gathers(r):
                cp.start()

        def issue_idx():
            idx_copy(u, n + 4).start()

        if u < 2:
            @pl.when(m > 0)
            def _():
                out_copy(r, base).wait()
            refill()

            @pl.when(m < NITER - 1)
            def _():
                issue_idx()
        else:
            @pl.when(m < NITER - 1)
            def _():
                out_copy(r, base).wait()
                refill()
                issue_idx()

    def body(m, _):
        for u in range(4):
            chunk(u, m)
        return 0

    lax.fori_loop(0, NITER, body, 0)

    for u in range(4):
        out_copy(u, wid * TPW).wait()


@jax.jit
def _run(idxc, token_table, cont_packed, pe_packed):
    kern = pl.kernel(
        _body,
        out_type=jax.ShapeDtypeStruct((N, E), jnp.float32),
        mesh=plsc.VectorSubcoreMesh(core_axis_name="c", subcore_axis_name="s"),
        compiler_params=pltpu.CompilerParams(use_tc_tiling_on_sc=False),
        scratch_types=(
            [pltpu.VMEM((LSEQ, E // 2), jnp.int32)]
            + [pltpu.VMEM((KP, T), jnp.int32)] * 4
            + [pltpu.VMEM((T, E), jnp.float32)] * 4
            + [pltpu.VMEM((K, T, E // 2), jnp.int32)] * 4
            + [pltpu.SemaphoreType.DMA] * 12
        ),
    )
    return kern(idxc, token_table, cont_packed, pe_packed)


def kernel(sequence, c_sequence, token_table, content_table, pe):
    tok = sequence.astype(jnp.int32).reshape(NW, NCHUNK, 1, T)
    con = (c_sequence.astype(jnp.int32)
           .reshape(NW, NCHUNK, T, K).transpose(0, 1, 3, 2))
    idxc = jnp.concatenate([tok, con], axis=2)
    cont_packed = _pack_table(content_table)
    pe_packed = _pack_table(pe[0, :LSEQ])
    out = _run(idxc, token_table, cont_packed, pe_packed)
    return out.reshape(B, LSEQ, E)
